# Initial kernel scaffold; baseline (speedup 1.0000x reference)
import jax
import jax.numpy as jnp
from jax.experimental import pallas as pl


def kernel(x, states, edge_index, edge_weight, W_out):
    raise NotImplementedError("write your pallas kernel here")


# speedup vs baseline: 26.1385x; 26.1385x over previous
import functools

import jax
import jax.numpy as jnp
from jax import lax
from jax.experimental import pallas as pl
from jax.experimental.pallas import tpu as pltpu
from jax.experimental.pallas import tpu_sc as plsc

N_NODES = 100000
N_EDGES = 3200000
INPUT_SIZE = 512
OUTPUT_SIZE = 128
BATCH = 8

NC = 2
NS = 16
NW = NC * NS
EDGES_PER_W = N_EDGES // NW
CHUNK = 800
NCHUNK = EDGES_PER_W // CHUNK
ROWS_PER_TILE = 6272
N_PAD = ROWS_PER_TILE * NS
B_PAD = 16


def _sc_body(states_hbm, src_hbm, dst_hbm, w_hbm, zeros_hbm, out_hbm,
             agg_sh, src_v, dst_v, w_v, rows_v, sem):
    c = lax.axis_index("c")
    s = lax.axis_index("s")
    wid = s * NC + c

    r0 = s * ROWS_PER_TILE
    pltpu.sync_copy(zeros_hbm.at[pl.ds(r0, ROWS_PER_TILE)],
                    agg_sh.at[pl.ds(r0, ROWS_PER_TILE)])
    plsc.subcore_barrier()

    wperm = [jnp.full((16,), t, jnp.int32) for t in range(16)]
    _gd = lax.GatherDimensionNumbers(offset_dims=(), collapsed_slice_dims=(0,),
                                     start_index_map=(0,))

    def _lane_bcast(vec16, idx16):
        return lax.gather(vec16, idx16[:, None], _gd, (1,),
                          mode=lax.GatherScatterMode.PROMISE_IN_BOUNDS)

    def chunk_body(k, carry):
        off = wid * EDGES_PER_W + k * CHUNK
        pltpu.sync_copy(src_hbm.at[pl.ds(off, CHUNK)], src_v)
        pltpu.sync_copy(dst_hbm.at[pl.ds(off, CHUNK)], dst_v)
        pltpu.sync_copy(w_hbm.at[pl.ds(off, CHUNK)], w_v)
        pltpu.async_copy(states_hbm.at[src_v], rows_v, sem).wait()

        def mul_body(j, carry2):
            w16 = w_v[pl.ds(j * 16, 16)]
            for t in range(16):
                e = j * 16 + t
                rows_v[e] = rows_v[e] * _lane_bcast(w16, wperm[t])
            return carry2

        lax.fori_loop(0, CHUNK // 16, mul_body, 0, unroll=False)
        pltpu.sync_copy(rows_v, agg_sh.at[dst_v], add=True)
        return carry

    lax.fori_loop(0, NCHUNK, chunk_body, 0, unroll=False)

    plsc.subcore_barrier()
    pltpu.sync_copy(agg_sh.at[pl.ds(r0, ROWS_PER_TILE)],
                    out_hbm.at[c, pl.ds(r0, ROWS_PER_TILE)])


_sc_aggregate = pl.kernel(
    _sc_body,
    out_type=jax.ShapeDtypeStruct((NC, N_PAD, B_PAD), jnp.float32),
    mesh=plsc.VectorSubcoreMesh(core_axis_name="c", subcore_axis_name="s"),
    scratch_types=[
        pltpu.VMEM_SHARED((N_PAD, B_PAD), jnp.float32),
        pltpu.VMEM((CHUNK,), jnp.int32),
        pltpu.VMEM((CHUNK,), jnp.int32),
        pltpu.VMEM((CHUNK,), jnp.float32),
        pltpu.VMEM((CHUNK, B_PAD), jnp.float32),
        pltpu.SemaphoreType.DMA,
    ],
    compiler_params=pltpu.CompilerParams(use_tc_tiling_on_sc=False),
)

_ROWS_BLK = 2000
_N_BLOCKS = N_NODES // _ROWS_BLK


def _tc_body(p_ref, xp_ref, w_ref, y_ref, ns_ref):
    i = pl.program_id(0)
    a = p_ref[0, :, :BATCH] + p_ref[1, :, :BATCH]
    a = a + jnp.where(i == 0, 1.0, 0.0) * xp_ref[...]
    t = jnp.tanh(a)
    ns_ref[...] = t
    contrib = lax.dot_general(t, w_ref[...], (((0,), (0,)), ((), ())),
                              preferred_element_type=jnp.float32)

    @pl.when(i == 0)
    def _():
        y_ref[...] = contrib

    @pl.when(i != 0)
    def _():
        y_ref[...] = y_ref[...] + contrib


_tc_finish = pl.pallas_call(
    _tc_body,
    grid=(_N_BLOCKS,),
    in_specs=[
        pl.BlockSpec((NC, _ROWS_BLK, B_PAD), lambda i: (0, i, 0)),
        pl.BlockSpec((_ROWS_BLK, BATCH), lambda i: (0, 0)),
        pl.BlockSpec((_ROWS_BLK, OUTPUT_SIZE), lambda i: (i, 0)),
    ],
    out_specs=[
        pl.BlockSpec((BATCH, OUTPUT_SIZE), lambda i: (0, 0)),
        pl.BlockSpec((_ROWS_BLK, BATCH), lambda i: (i, 0)),
    ],
    out_shape=[
        jax.ShapeDtypeStruct((BATCH, OUTPUT_SIZE), jnp.float32),
        jax.ShapeDtypeStruct((N_NODES, BATCH), jnp.float32),
    ],
)


@jax.jit
def kernel(x, states, edge_index, edge_weight, W_out):
    states_t = jnp.zeros((N_NODES, B_PAD), jnp.float32).at[:, :BATCH].set(states.T)
    src = edge_index[0]
    dst = edge_index[1]
    zeros = jnp.zeros((N_PAD, B_PAD), jnp.float32)
    parts = _sc_aggregate(states_t, src, dst, edge_weight, zeros)
    x_pad = jnp.zeros((_ROWS_BLK, BATCH), jnp.float32).at[:INPUT_SIZE].set(x.T)
    y, ns_t = _tc_finish(parts[:, :N_NODES], x_pad, W_out)
    return (y, ns_t.T)


# speedup vs baseline: 44.0908x; 1.6868x over previous
import functools

import jax
import jax.numpy as jnp
from jax import lax
from jax.experimental import pallas as pl
from jax.experimental.pallas import tpu as pltpu
from jax.experimental.pallas import tpu_sc as plsc

N_NODES = 100000
N_EDGES = 3200000
INPUT_SIZE = 512
OUTPUT_SIZE = 128
BATCH = 8

NC = 2
NS = 16
NW = NC * NS
EDGES_PER_W = N_EDGES // NW
CHUNK = 800
NCHUNK = EDGES_PER_W // CHUNK
ROWS_PER_TILE = 6272
N_PAD = ROWS_PER_TILE * NS
B_PAD = 16


_ZROWS = 784


def _sc_body(states_hbm, src_hbm, dst_hbm, w_hbm, out_hbm, agg_sh,
             src_a, dst_a, w_a, rows_a, isem_a, gsem_a,
             src_b, dst_b, w_b, rows_b, isem_b, gsem_b):
    c = lax.axis_index("c")
    s = lax.axis_index("s")
    wid = s * NC + c
    base = wid * EDGES_PER_W
    r0 = s * ROWS_PER_TILE

    zero16 = jnp.zeros((16,), jnp.float32)

    def zb(e, carry):
        rows_a[e] = zero16
        return carry

    lax.fori_loop(0, _ZROWS, zb, 0, unroll=False)
    for i in range(ROWS_PER_TILE // _ZROWS):
        pltpu.sync_copy(rows_a.at[pl.ds(0, _ZROWS)],
                        agg_sh.at[pl.ds(r0 + i * _ZROWS, _ZROWS)])
    plsc.subcore_barrier()

    wperm = [jnp.full((16,), t, jnp.int32) for t in range(16)]
    _gd = lax.GatherDimensionNumbers(offset_dims=(), collapsed_slice_dims=(0,),
                                     start_index_map=(0,))

    def _lane_bcast(vec16, idx16):
        return lax.gather(vec16, idx16[:, None], _gd, (1,),
                          mode=lax.GatherScatterMode.PROMISE_IN_BOUNDS)

    bufs = ((src_a, dst_a, w_a, rows_a, isem_a, gsem_a),
            (src_b, dst_b, w_b, rows_b, isem_b, gsem_b))

    def issue_idx(k, buf):
        src_v, dst_v, w_v, _, isem, _ = buf
        off = base + k * CHUNK
        pltpu.async_copy(src_hbm.at[pl.ds(off, CHUNK)], src_v, isem)
        pltpu.async_copy(dst_hbm.at[pl.ds(off, CHUNK)], dst_v, isem)
        pltpu.async_copy(w_hbm.at[pl.ds(off, CHUNK)], w_v, isem)

    def wait_idx(k, buf):
        src_v, dst_v, w_v, _, isem, _ = buf
        off = base + k * CHUNK
        pltpu.make_async_copy(src_hbm.at[pl.ds(off, CHUNK)], src_v, isem).wait()
        pltpu.make_async_copy(dst_hbm.at[pl.ds(off, CHUNK)], dst_v, isem).wait()
        pltpu.make_async_copy(w_hbm.at[pl.ds(off, CHUNK)], w_v, isem).wait()

    def issue_gather(buf):
        src_v, _, _, rows_v, _, gsem = buf
        pltpu.async_copy(states_hbm.at[src_v], rows_v, gsem)

    def wait_gather(buf):
        src_v, _, _, rows_v, _, gsem = buf
        pltpu.make_async_copy(states_hbm.at[src_v], rows_v, gsem).wait()

    def mul_scatter(buf):
        _, dst_v, w_v, rows_v, _, _ = buf

        def mul_body(j, carry2):
            w16 = w_v[pl.ds(j * 16, 16)]
            for t in range(16):
                e = j * 16 + t
                rows_v[e] = rows_v[e] * _lane_bcast(w16, wperm[t])
            return carry2

        lax.fori_loop(0, CHUNK // 16, mul_body, 0, unroll=False)
        pltpu.sync_copy(rows_v, agg_sh.at[dst_v], add=True)

    def body(k, cur, nxt, steady):
        wait_gather(cur)
        wait_idx(k + 1, nxt)
        issue_gather(nxt)
        mul_scatter(cur)
        if steady:
            @pl.when(k + 2 < NCHUNK)
            def _():
                issue_idx(k + 2, cur)

    issue_idx(0, bufs[0])
    wait_idx(0, bufs[0])
    issue_gather(bufs[0])
    issue_idx(1, bufs[1])

    def pair_body(t, carry):
        k = 2 * t
        body(k, bufs[0], bufs[1], True)
        body(k + 1, bufs[1], bufs[0], True)
        return carry

    lax.fori_loop(0, (NCHUNK - 1) // 2, pair_body, 0, unroll=False)
    wait_gather(bufs[0])
    mul_scatter(bufs[0])

    plsc.subcore_barrier()
    pltpu.sync_copy(agg_sh.at[pl.ds(r0, ROWS_PER_TILE)],
                    out_hbm.at[c, pl.ds(r0, ROWS_PER_TILE)])


_sc_aggregate = pl.kernel(
    _sc_body,
    out_type=jax.ShapeDtypeStruct((NC, N_PAD, B_PAD), jnp.float32),
    mesh=plsc.VectorSubcoreMesh(core_axis_name="c", subcore_axis_name="s"),
    scratch_types=[pltpu.VMEM_SHARED((N_PAD, B_PAD), jnp.float32)] + 2 * [
        pltpu.VMEM((CHUNK,), jnp.int32),
        pltpu.VMEM((CHUNK,), jnp.int32),
        pltpu.VMEM((CHUNK,), jnp.float32),
        pltpu.VMEM((CHUNK, B_PAD), jnp.float32),
        pltpu.SemaphoreType.DMA,
        pltpu.SemaphoreType.DMA,
    ],
    compiler_params=pltpu.CompilerParams(use_tc_tiling_on_sc=False),
)

_ROWS_BLK = 2000
_N_BLOCKS = N_NODES // _ROWS_BLK


def _tc_body(p_ref, xp_ref, w_ref, y_ref, ns_ref):
    i = pl.program_id(0)
    a = p_ref[0, :, :BATCH] + p_ref[1, :, :BATCH]
    a = a + jnp.where(i == 0, 1.0, 0.0) * xp_ref[...]
    t = jnp.tanh(a)
    ns_ref[...] = t
    contrib = lax.dot_general(t, w_ref[...], (((0,), (0,)), ((), ())),
                              preferred_element_type=jnp.float32)

    @pl.when(i == 0)
    def _():
        y_ref[...] = contrib

    @pl.when(i != 0)
    def _():
        y_ref[...] = y_ref[...] + contrib


_tc_finish = pl.pallas_call(
    _tc_body,
    grid=(_N_BLOCKS,),
    in_specs=[
        pl.BlockSpec((NC, _ROWS_BLK, B_PAD), lambda i: (0, i, 0)),
        pl.BlockSpec((_ROWS_BLK, BATCH), lambda i: (0, 0)),
        pl.BlockSpec((_ROWS_BLK, OUTPUT_SIZE), lambda i: (i, 0)),
    ],
    out_specs=[
        pl.BlockSpec((BATCH, OUTPUT_SIZE), lambda i: (0, 0)),
        pl.BlockSpec((_ROWS_BLK, BATCH), lambda i: (i, 0)),
    ],
    out_shape=[
        jax.ShapeDtypeStruct((BATCH, OUTPUT_SIZE), jnp.float32),
        jax.ShapeDtypeStruct((N_NODES, BATCH), jnp.float32),
    ],
)


@jax.jit
def kernel(x, states, edge_index, edge_weight, W_out):
    states_t = jnp.zeros((N_NODES, B_PAD), jnp.float32).at[:, :BATCH].set(states.T)
    src = edge_index[0]
    dst = edge_index[1]
    parts = _sc_aggregate(states_t, src, dst, edge_weight)
    x_pad = jnp.zeros((_ROWS_BLK, BATCH), jnp.float32).at[:INPUT_SIZE].set(x.T)
    y, ns_t = _tc_finish(parts, x_pad, W_out)
    return (y, ns_t.T)


# speedup vs baseline: 48.7212x; 1.1050x over previous
import functools

import jax
import jax.numpy as jnp
from jax import lax
from jax.experimental import pallas as pl
from jax.experimental.pallas import tpu as pltpu
from jax.experimental.pallas import tpu_sc as plsc

N_NODES = 100000
N_EDGES = 3200000
INPUT_SIZE = 512
OUTPUT_SIZE = 128
BATCH = 8

NC = 2
NS = 16
NW = NC * NS
EDGES_PER_W = N_EDGES // NW
CHUNK = 800
NCHUNK = EDGES_PER_W // CHUNK
ROWS_PER_TILE = 6272
N_PAD = ROWS_PER_TILE * NS
B_PAD = 16


_ZROWS = 784


def _sc_body(states_hbm, src_hbm, dst_hbm, w_hbm, out_hbm, agg_sh,
             src_a, dst_a, w_a, rows_a, isem_a, gsem_a,
             src_b, dst_b, w_b, rows_b, isem_b, gsem_b):
    c = lax.axis_index("c")
    s = lax.axis_index("s")
    wid = s * NC + c
    base = wid * EDGES_PER_W
    r0 = s * ROWS_PER_TILE

    zero16 = jnp.zeros((16,), jnp.float32)

    def zb(e, carry):
        rows_a[e] = zero16
        return carry

    lax.fori_loop(0, _ZROWS, zb, 0, unroll=False)
    for i in range(ROWS_PER_TILE // _ZROWS):
        pltpu.sync_copy(rows_a.at[pl.ds(0, _ZROWS)],
                        agg_sh.at[pl.ds(r0 + i * _ZROWS, _ZROWS)])
    plsc.subcore_barrier()

    wperm = [jnp.full((16,), t, jnp.int32) for t in range(16)]
    _gd = lax.GatherDimensionNumbers(offset_dims=(), collapsed_slice_dims=(0,),
                                     start_index_map=(0,))

    def _lane_bcast(vec16, idx16):
        return lax.gather(vec16, idx16[:, None], _gd, (1,),
                          mode=lax.GatherScatterMode.PROMISE_IN_BOUNDS)

    bufs = ((src_a, dst_a, w_a, rows_a, isem_a, gsem_a),
            (src_b, dst_b, w_b, rows_b, isem_b, gsem_b))

    def issue_idx(k, buf):
        src_v, dst_v, w_v, _, isem, _ = buf
        off = base + k * CHUNK
        pltpu.async_copy(src_hbm.at[pl.ds(off, CHUNK)], src_v, isem)
        pltpu.async_copy(dst_hbm.at[pl.ds(off, CHUNK)], dst_v, isem)
        pltpu.async_copy(w_hbm.at[pl.ds(off, CHUNK)], w_v, isem)

    def wait_idx(k, buf):
        src_v, dst_v, w_v, _, isem, _ = buf
        off = base + k * CHUNK
        pltpu.make_async_copy(src_hbm.at[pl.ds(off, CHUNK)], src_v, isem).wait()
        pltpu.make_async_copy(dst_hbm.at[pl.ds(off, CHUNK)], dst_v, isem).wait()
        pltpu.make_async_copy(w_hbm.at[pl.ds(off, CHUNK)], w_v, isem).wait()

    def issue_gather(buf):
        src_v, _, _, rows_v, _, gsem = buf
        pltpu.async_copy(states_hbm.at[src_v], rows_v, gsem)

    def wait_gather(buf):
        src_v, _, _, rows_v, _, gsem = buf
        pltpu.make_async_copy(states_hbm.at[src_v], rows_v, gsem).wait()

    def mul_scatter(buf):
        _, dst_v, w_v, rows_v, _, _ = buf

        def mul_body(j, carry2):
            w16 = w_v[pl.ds(j * 16, 16)]
            for t in range(16):
                e = j * 16 + t
                rows_v[e] = rows_v[e] * _lane_bcast(w16, wperm[t])
            return carry2

        lax.fori_loop(0, CHUNK // 16, mul_body, 0, unroll=False)
        pltpu.sync_copy(rows_v, agg_sh.at[dst_v], add=True)

    def body(k, cur, nxt, steady):
        wait_gather(cur)
        wait_idx(k + 1, nxt)
        issue_gather(nxt)
        mul_scatter(cur)
        if steady:
            @pl.when(k + 2 < NCHUNK)
            def _():
                issue_idx(k + 2, cur)

    issue_idx(0, bufs[0])
    wait_idx(0, bufs[0])
    issue_gather(bufs[0])
    issue_idx(1, bufs[1])

    def pair_body(t, carry):
        k = 2 * t
        body(k, bufs[0], bufs[1], True)
        body(k + 1, bufs[1], bufs[0], True)
        return carry

    lax.fori_loop(0, (NCHUNK - 1) // 2, pair_body, 0, unroll=False)
    wait_gather(bufs[0])
    mul_scatter(bufs[0])

    plsc.subcore_barrier()
    pltpu.sync_copy(agg_sh.at[pl.ds(r0, ROWS_PER_TILE)],
                    out_hbm.at[c, pl.ds(r0, ROWS_PER_TILE)])


_sc_aggregate = pl.kernel(
    _sc_body,
    out_type=jax.ShapeDtypeStruct((NC, N_PAD, B_PAD), jnp.float32),
    mesh=plsc.VectorSubcoreMesh(core_axis_name="c", subcore_axis_name="s"),
    scratch_types=[pltpu.VMEM_SHARED((N_PAD, B_PAD), jnp.float32)] + 2 * [
        pltpu.VMEM((CHUNK,), jnp.int32),
        pltpu.VMEM((CHUNK,), jnp.int32),
        pltpu.VMEM((CHUNK,), jnp.float32),
        pltpu.VMEM((CHUNK, B_PAD), jnp.float32),
        pltpu.SemaphoreType.DMA,
        pltpu.SemaphoreType.DMA,
    ],
    compiler_params=pltpu.CompilerParams(use_tc_tiling_on_sc=False),
)

_ROWS_BLK = 2048
_N_BLOCKS = N_PAD // _ROWS_BLK


def _pre_body(s_ref, o_ref):
    st = s_ref[...]
    o_ref[...] = jnp.concatenate(
        [st.T, jnp.zeros((_ROWS_BLK, B_PAD - BATCH), jnp.float32)], axis=1)


_tc_pre = pl.pallas_call(
    _pre_body,
    grid=(_N_BLOCKS,),
    in_specs=[pl.BlockSpec((BATCH, _ROWS_BLK), lambda i: (0, i))],
    out_specs=pl.BlockSpec((_ROWS_BLK, B_PAD), lambda i: (i, 0)),
    out_shape=jax.ShapeDtypeStruct((N_NODES, B_PAD), jnp.float32),
)


def _tc_body(p_ref, xp_ref, w_ref, y_ref, ns_ref):
    i = pl.program_id(0)
    a = p_ref[0, :, :BATCH] + p_ref[1, :, :BATCH]
    a = a + jnp.where(i == 0, 1.0, 0.0) * xp_ref[...]
    t = jnp.tanh(a)
    ns_ref[...] = t.T
    rows = i * _ROWS_BLK + lax.broadcasted_iota(jnp.int32, (_ROWS_BLK, 1), 0)
    wm = jnp.where(rows < N_NODES, w_ref[...], 0.0)
    contrib = lax.dot_general(t, wm, (((0,), (0,)), ((), ())),
                              preferred_element_type=jnp.float32)

    @pl.when(i == 0)
    def _():
        y_ref[...] = contrib

    @pl.when(i != 0)
    def _():
        y_ref[...] = y_ref[...] + contrib


_tc_finish = pl.pallas_call(
    _tc_body,
    grid=(_N_BLOCKS,),
    in_specs=[
        pl.BlockSpec((NC, _ROWS_BLK, B_PAD), lambda i: (0, i, 0)),
        pl.BlockSpec((_ROWS_BLK, BATCH), lambda i: (0, 0)),
        pl.BlockSpec((_ROWS_BLK, OUTPUT_SIZE), lambda i: (i, 0)),
    ],
    out_specs=[
        pl.BlockSpec((BATCH, OUTPUT_SIZE), lambda i: (0, 0)),
        pl.BlockSpec((BATCH, _ROWS_BLK), lambda i: (0, i)),
    ],
    out_shape=[
        jax.ShapeDtypeStruct((BATCH, OUTPUT_SIZE), jnp.float32),
        jax.ShapeDtypeStruct((BATCH, N_NODES), jnp.float32),
    ],
)


@jax.jit
def kernel(x, states, edge_index, edge_weight, W_out):
    states_t = _tc_pre(states)
    src = edge_index[0]
    dst = edge_index[1]
    parts = _sc_aggregate(states_t, src, dst, edge_weight)
    x_pad = jnp.zeros((_ROWS_BLK, BATCH), jnp.float32).at[:INPUT_SIZE].set(x.T)
    y, ns = _tc_finish(parts, x_pad, W_out)
    return (y, ns)


# speedup vs baseline: 52.9866x; 1.0875x over previous
import functools

import jax
import jax.numpy as jnp
from jax import lax
from jax.experimental import pallas as pl
from jax.experimental.pallas import tpu as pltpu
from jax.experimental.pallas import tpu_sc as plsc

N_NODES = 100000
N_EDGES = 3200000
INPUT_SIZE = 512
OUTPUT_SIZE = 128
BATCH = 8

NC = 2
NS = 16
NW = NC * NS
EDGES_PER_W = N_EDGES // NW
CHUNK = 800
NCHUNK = EDGES_PER_W // CHUNK
ROWS_PER_TILE = 6272
N_PAD = ROWS_PER_TILE * NS
B_PAD = 16


_ZROWS = 784


def _sc_body(states_hbm, src_hbm, dst_hbm, w_hbm, out_hbm, agg_sh,
             src_a, dst_a, w_a, rows_a, isem_a, dwsem_a, gsem_a, scsem_a,
             src_b, dst_b, w_b, rows_b, isem_b, dwsem_b, gsem_b, scsem_b):
    c = lax.axis_index("c")
    s = lax.axis_index("s")
    wid = s * NC + c
    base = wid * EDGES_PER_W
    r0 = s * ROWS_PER_TILE

    zero16 = jnp.zeros((16,), jnp.float32)

    def zb(e, carry):
        rows_a[e] = zero16
        return carry

    lax.fori_loop(0, _ZROWS, zb, 0, unroll=False)
    for i in range(ROWS_PER_TILE // _ZROWS):
        pltpu.sync_copy(rows_a.at[pl.ds(0, _ZROWS)],
                        agg_sh.at[pl.ds(r0 + i * _ZROWS, _ZROWS)])
    plsc.subcore_barrier()

    wperm = [jnp.full((16,), t, jnp.int32) for t in range(16)]
    _gd = lax.GatherDimensionNumbers(offset_dims=(), collapsed_slice_dims=(0,),
                                     start_index_map=(0,))

    def _lane_bcast(vec16, idx16):
        return lax.gather(vec16, idx16[:, None], _gd, (1,),
                          mode=lax.GatherScatterMode.PROMISE_IN_BOUNDS)

    bufs = ((src_a, dst_a, w_a, rows_a, isem_a, dwsem_a, gsem_a, scsem_a),
            (src_b, dst_b, w_b, rows_b, isem_b, dwsem_b, gsem_b, scsem_b))

    def issue_src(k, buf):
        src_v, _, _, _, isem, _, _, _ = buf
        off = base + k * CHUNK
        pltpu.async_copy(src_hbm.at[pl.ds(off, CHUNK)], src_v, isem)

    def wait_src(k, buf):
        src_v, _, _, _, isem, _, _, _ = buf
        off = base + k * CHUNK
        pltpu.make_async_copy(src_hbm.at[pl.ds(off, CHUNK)], src_v, isem).wait()

    def issue_dstw(k, buf):
        _, dst_v, w_v, _, _, dwsem, _, _ = buf
        off = base + k * CHUNK
        pltpu.async_copy(dst_hbm.at[pl.ds(off, CHUNK)], dst_v, dwsem)
        pltpu.async_copy(w_hbm.at[pl.ds(off, CHUNK)], w_v, dwsem)

    def wait_dstw(k, buf):
        _, dst_v, w_v, _, _, dwsem, _, _ = buf
        off = base + k * CHUNK
        pltpu.make_async_copy(dst_hbm.at[pl.ds(off, CHUNK)], dst_v, dwsem).wait()
        pltpu.make_async_copy(w_hbm.at[pl.ds(off, CHUNK)], w_v, dwsem).wait()

    def issue_gather(buf):
        src_v, _, _, rows_v, _, _, gsem, _ = buf
        pltpu.async_copy(states_hbm.at[src_v], rows_v, gsem)

    def wait_gather(buf):
        src_v, _, _, rows_v, _, _, gsem, _ = buf
        pltpu.make_async_copy(states_hbm.at[src_v], rows_v, gsem).wait()

    def issue_scatter(buf):
        _, dst_v, _, rows_v, _, _, _, scsem = buf
        pltpu.async_copy(rows_v, agg_sh.at[dst_v], scsem, add=True)

    def wait_scatter(buf):
        _, dst_v, _, rows_v, _, _, _, scsem = buf
        pltpu.make_async_copy(rows_v, agg_sh.at[dst_v], scsem).wait()

    def multiply(buf):
        _, _, w_v, rows_v, _, _, _, _ = buf

        def mul_body(j, carry2):
            w16 = w_v[pl.ds(j * 16, 16)]
            for t in range(16):
                e = j * 16 + t
                rows_v[e] = rows_v[e] * _lane_bcast(w16, wperm[t])
            return carry2

        lax.fori_loop(0, CHUNK // 16, mul_body, 0, unroll=False)

    def body(k, cur, nxt, first=False, last=False):
        wait_gather(cur)
        if not last:
            wait_src(k + 1, nxt)
        if not first:
            wait_scatter(nxt)
        if not last:
            issue_gather(nxt)
            issue_dstw(k + 1, nxt)
        wait_dstw(k, cur)
        multiply(cur)
        issue_scatter(cur)
        if not last:
            @pl.when(k + 2 < NCHUNK)
            def _():
                issue_src(k + 2, cur)

    issue_src(0, bufs[0])
    wait_src(0, bufs[0])
    issue_gather(bufs[0])
    issue_dstw(0, bufs[0])
    issue_src(1, bufs[1])

    def pair_body(t, carry):
        k = 2 * t

        @pl.when(k == 0)
        def _():
            body(0, bufs[0], bufs[1], first=True)

        @pl.when(k > 0)
        def _():
            body(k, bufs[0], bufs[1])

        body(k + 1, bufs[1], bufs[0])
        return carry

    lax.fori_loop(0, (NCHUNK - 1) // 2, pair_body, 0, unroll=False)
    body(NCHUNK - 1, bufs[0], bufs[1], last=True)
    wait_scatter(bufs[0])

    plsc.subcore_barrier()
    pltpu.sync_copy(agg_sh.at[pl.ds(r0, ROWS_PER_TILE)],
                    out_hbm.at[c, pl.ds(r0, ROWS_PER_TILE)])


_sc_aggregate = pl.kernel(
    _sc_body,
    out_type=jax.ShapeDtypeStruct((NC, N_PAD, B_PAD), jnp.float32),
    mesh=plsc.VectorSubcoreMesh(core_axis_name="c", subcore_axis_name="s"),
    scratch_types=[pltpu.VMEM_SHARED((N_PAD, B_PAD), jnp.float32)] + 2 * [
        pltpu.VMEM((CHUNK,), jnp.int32),
        pltpu.VMEM((CHUNK,), jnp.int32),
        pltpu.VMEM((CHUNK,), jnp.float32),
        pltpu.VMEM((CHUNK, B_PAD), jnp.float32),
        pltpu.SemaphoreType.DMA,
        pltpu.SemaphoreType.DMA,
        pltpu.SemaphoreType.DMA,
        pltpu.SemaphoreType.DMA,
    ],
    compiler_params=pltpu.CompilerParams(use_tc_tiling_on_sc=False),
)

_ROWS_BLK = 2048
_N_BLOCKS = N_PAD // _ROWS_BLK


def _pre_body(s_ref, o_ref):
    st = s_ref[...]
    o_ref[...] = jnp.concatenate(
        [st.T, jnp.zeros((_ROWS_BLK, B_PAD - BATCH), jnp.float32)], axis=1)


_tc_pre = pl.pallas_call(
    _pre_body,
    grid=(_N_BLOCKS,),
    in_specs=[pl.BlockSpec((BATCH, _ROWS_BLK), lambda i: (0, i))],
    out_specs=pl.BlockSpec((_ROWS_BLK, B_PAD), lambda i: (i, 0)),
    out_shape=jax.ShapeDtypeStruct((N_NODES, B_PAD), jnp.float32),
)


def _tc_body(p_ref, xp_ref, w_ref, y_ref, ns_ref):
    i = pl.program_id(0)
    a = p_ref[0, :, :BATCH] + p_ref[1, :, :BATCH]
    a = a + jnp.where(i == 0, 1.0, 0.0) * xp_ref[...]
    t = jnp.tanh(a)
    ns_ref[...] = t.T
    rows = i * _ROWS_BLK + lax.broadcasted_iota(jnp.int32, (_ROWS_BLK, 1), 0)
    wm = jnp.where(rows < N_NODES, w_ref[...], 0.0)
    contrib = lax.dot_general(t, wm, (((0,), (0,)), ((), ())),
                              preferred_element_type=jnp.float32)

    @pl.when(i == 0)
    def _():
        y_ref[...] = contrib

    @pl.when(i != 0)
    def _():
        y_ref[...] = y_ref[...] + contrib


_tc_finish = pl.pallas_call(
    _tc_body,
    grid=(_N_BLOCKS,),
    in_specs=[
        pl.BlockSpec((NC, _ROWS_BLK, B_PAD), lambda i: (0, i, 0)),
        pl.BlockSpec((_ROWS_BLK, BATCH), lambda i: (0, 0)),
        pl.BlockSpec((_ROWS_BLK, OUTPUT_SIZE), lambda i: (i, 0)),
    ],
    out_specs=[
        pl.BlockSpec((BATCH, OUTPUT_SIZE), lambda i: (0, 0)),
        pl.BlockSpec((BATCH, _ROWS_BLK), lambda i: (0, i)),
    ],
    out_shape=[
        jax.ShapeDtypeStruct((BATCH, OUTPUT_SIZE), jnp.float32),
        jax.ShapeDtypeStruct((BATCH, N_NODES), jnp.float32),
    ],
)


@jax.jit
def kernel(x, states, edge_index, edge_weight, W_out):
    states_t = _tc_pre(states)
    src = edge_index[0]
    dst = edge_index[1]
    parts = _sc_aggregate(states_t, src, dst, edge_weight)
    x_pad = jnp.zeros((_ROWS_BLK, BATCH), jnp.float32).at[:INPUT_SIZE].set(x.T)
    y, ns = _tc_finish(parts, x_pad, W_out)
    return (y, ns)
